# Initial kernel scaffold; baseline (speedup 1.0000x reference)
import jax
import jax.numpy as jnp
from jax.experimental import pallas as pl


def kernel(x, edge_index, W1, b1, W2, b2, W3, b3):
    raise NotImplementedError("write your pallas kernel here")


# speedup vs baseline: 5.0889x; 5.0889x over previous
import functools

import jax
import jax.numpy as jnp
from jax import lax
from jax.experimental import pallas as pl
from jax.experimental.pallas import tpu as pltpu
from jax.experimental.pallas import tpu_sc as plsc

N_NODES = 10000
D = 128
N_EDGES = 320000
NC = 2
NS = 16
NW = NC * NS
CH = 128
NCH = 79
E_PAD = NW * NCH * CH
DUMP = N_NODES
ACC_ROWS = N_NODES + 16
ROWS_PER_TILE = 624


def _sc_agg_body(x_hbm, src_hbm, dst_hbm, out_hbm, src_v, dst_v, rows_v, acc, sem):
  c = lax.axis_index("c")
  s = lax.axis_index("s")
  wid = s * NC + c

  pltpu.sync_copy(src_hbm.at[wid], src_v)
  pltpu.sync_copy(dst_hbm.at[wid], dst_v)

  start = s * ROWS_PER_TILE
  pltpu.sync_copy(x_hbm.at[pl.ds(start, ROWS_PER_TILE)],
                  acc.at[pl.ds(start, ROWS_PER_TILE)])

  @pl.when(s == NS - 1)
  def _():
    pltpu.sync_copy(x_hbm.at[pl.ds(NS * ROWS_PER_TILE, N_NODES - NS * ROWS_PER_TILE)],
                    acc.at[pl.ds(NS * ROWS_PER_TILE, N_NODES - NS * ROWS_PER_TILE)])

  plsc.subcore_barrier()

  @pl.loop(0, NCH)
  def _(j):
    pltpu.async_copy(x_hbm.at[src_v.at[j]], rows_v, sem).wait()
    pltpu.sync_copy(rows_v, acc.at[dst_v.at[j]], add=True)

  plsc.subcore_barrier()

  pltpu.sync_copy(acc.at[pl.ds(start, ROWS_PER_TILE)],
                  out_hbm.at[c, pl.ds(start, ROWS_PER_TILE)])

  @pl.when(s == NS - 1)
  def _():
    pltpu.sync_copy(acc.at[pl.ds(NS * ROWS_PER_TILE, N_NODES - NS * ROWS_PER_TILE)],
                    out_hbm.at[c, pl.ds(NS * ROWS_PER_TILE, N_NODES - NS * ROWS_PER_TILE)])


_sc_agg = functools.partial(
    pl.kernel,
    out_type=jax.ShapeDtypeStruct((NC, N_NODES, D), jnp.float32),
    mesh=plsc.VectorSubcoreMesh(
        core_axis_name="c", subcore_axis_name="s", num_cores=NC, num_subcores=NS),
    scratch_types=[
        pltpu.VMEM((NCH, CH), jnp.int32),
        pltpu.VMEM((NCH, CH), jnp.int32),
        pltpu.VMEM((CH, D), jnp.float32),
        pltpu.VMEM_SHARED((ACC_ROWS, D), jnp.float32),
        pltpu.SemaphoreType.DMA,
    ],
)(_sc_agg_body)


def _mlp_body(x_ref, p_ref, w1_ref, b1_ref, w2_ref, b2_ref, w3_ref, b3_ref, o_ref):
  h = p_ref[0] + p_ref[1] - x_ref[...]
  h = jnp.maximum(jnp.dot(h, w1_ref[...], preferred_element_type=jnp.float32)
                  + b1_ref[...], 0.0)
  h = jnp.maximum(jnp.dot(h, w2_ref[...], preferred_element_type=jnp.float32)
                  + b2_ref[...], 0.0)
  o_ref[...] = jnp.maximum(jnp.dot(h, w3_ref[...], preferred_element_type=jnp.float32)
                           + b3_ref[...], 0.0)


BLK = 1000


def _mlp(x, partials, w1t, b1, w2t, b2, w3t, b3):
  return pl.pallas_call(
      _mlp_body,
      grid=(N_NODES // BLK,),
      in_specs=[
          pl.BlockSpec((BLK, D), lambda i: (i, 0)),
          pl.BlockSpec((NC, BLK, D), lambda i: (0, i, 0)),
          pl.BlockSpec((D, D), lambda i: (0, 0)),
          pl.BlockSpec((1, D), lambda i: (0, 0)),
          pl.BlockSpec((D, D), lambda i: (0, 0)),
          pl.BlockSpec((1, D), lambda i: (0, 0)),
          pl.BlockSpec((D, D), lambda i: (0, 0)),
          pl.BlockSpec((1, D), lambda i: (0, 0)),
      ],
      out_specs=pl.BlockSpec((BLK, D), lambda i: (i, 0)),
      out_shape=jax.ShapeDtypeStruct((N_NODES, D), jnp.float32),
  )(x, partials, w1t, b1, w2t, b2, w3t, b3)


def kernel(x, edge_index, W1, b1, W2, b2, W3, b3):
  src = edge_index[0].astype(jnp.int32)
  dst = edge_index[1].astype(jnp.int32)
  pad = E_PAD - N_EDGES
  src_p = jnp.concatenate([src, jnp.zeros((pad,), jnp.int32)]).reshape(NW, NCH, CH)
  dst_p = jnp.concatenate([dst, jnp.full((pad,), DUMP, jnp.int32)]).reshape(NW, NCH, CH)
  partials = _sc_agg(x, src_p, dst_p)
  return _mlp(x, partials,
              W1.T, b1.reshape(1, D),
              W2.T, b2.reshape(1, D),
              W3.T, b3.reshape(1, D))
